# baseline (speedup 1.0000x reference)
import functools

import jax
import jax.numpy as jnp
from jax import lax
from jax.experimental import pallas as pl
from jax.experimental.pallas import tpu as pltpu
from jax.experimental.pallas import tpu_sc as plsc

VOCAB = 1000000
EMBED = 32
B = 4096
S = 200

NC = 2
NS = 16
NW = NC * NS

N = B * S
PER_W = N // NW
CHUNK = 128
NCH = PER_W // CHUNK
NBUF = 10
D = 5
NOUT = NCH // NBUF

_mesh = plsc.VectorSubcoreMesh(core_axis_name="c", subcore_axis_name="s")


@functools.partial(
    pl.kernel,
    mesh=_mesh,
    out_type=jax.ShapeDtypeStruct((N, EMBED), jnp.float32),
    scratch_types=[
        pltpu.VMEM((NCH, CHUNK), jnp.int32),
        pltpu.VMEM((NBUF, CHUNK, EMBED), jnp.float32),
        pltpu.SemaphoreType.DMA((NBUF,)),
        pltpu.SemaphoreType.DMA((NBUF,)),
    ],
    compiler_params=pltpu.CompilerParams(use_tc_tiling_on_sc=False),
)
def _emb_lookup(idx_hbm, table_hbm, out_hbm, idx_v, rows_v, gsem, ssem):
    wid = lax.axis_index("s") * NC + lax.axis_index("c")
    base = wid * PER_W

    pltpu.sync_copy(idx_hbm.at[wid], idx_v)

    def gather_start(j, b):
        pltpu.async_copy(table_hbm.at[idx_v.at[j]], rows_v.at[b], gsem.at[b])

    def gather_wait(j, b):
        pltpu.make_async_copy(
            table_hbm.at[idx_v.at[j]], rows_v.at[b], gsem.at[b]
        ).wait()

    def store_start(j, b):
        pltpu.async_copy(
            rows_v.at[b], out_hbm.at[pl.ds(base + j * CHUNK, CHUNK)], ssem.at[b]
        )

    def store_wait(j, b):
        pltpu.make_async_copy(
            rows_v.at[b], out_hbm.at[pl.ds(base + j * CHUNK, CHUNK)], ssem.at[b]
        ).wait()


    for b in range(NBUF):
        gather_start(b, b)

    for i in range(NBUF):
        gather_wait(i, i)
        store_start(i, i)
        if i >= D:
            store_wait(i - D, i - D)
            gather_start(i - D + NBUF, i - D)

    def outer(g, carry):
        for b in range(NBUF):
            i = g * NBUF + b
            gather_wait(i, b)
            store_start(i, b)
            bw = (b - D) % NBUF
            store_wait(i - D, bw)
            gather_start(i - D + NBUF, bw)
        return carry

    lax.fori_loop(1, NOUT - 1, outer, 0)

    for b in range(NBUF):
        i = (NOUT - 1) * NBUF + b
        gather_wait(i, b)
        store_start(i, b)
        bw = (b - D) % NBUF
        store_wait(i - D, bw)
        if i - D + NBUF < NCH:
            gather_start(i - D + NBUF, bw)

    for i in range(NCH - D, NCH):
        store_wait(i, i % NBUF)


def kernel(x, table):
    idx = x.reshape(NW, NCH, CHUNK).astype(jnp.int32)
    out = _emb_lookup(idx, table)
    return out.reshape(B, S, EMBED)
